# baseline (speedup 1.0000x reference)
import functools

import jax
import jax.numpy as jnp
from jax import lax
from jax.experimental import pallas as pl
from jax.experimental.pallas import tpu as pltpu
from jax.experimental.pallas import tpu_sc as plsc

D = 128
NC = 2
NS = 16
NW = NC * NS
CHUNK = 88


def _tc_forward_body(x_ref, mu_ref, ls_ref, eps_ref, sup_ref, kl_ref):
    ls = ls_ref[...]
    mu = mu_ref[...]
    sigma = jnp.exp(ls)
    w = mu + eps_ref[...] * sigma
    sup_ref[...] = jnp.dot(
        x_ref[...], w, preferred_element_type=jnp.float32
    ).astype(jnp.bfloat16)

    @pl.when(pl.program_id(0) == 0)
    def _():
        kl = 0.5 * (sigma * sigma + mu * mu - 2.0 * ls - 1.0)
        kl_ref[0, 0] = jnp.sum(kl)


def _tc_forward(x, mu, log_sigma, eps):
    n = x.shape[0]
    blk = 2000
    assert n % blk == 0
    return pl.pallas_call(
        _tc_forward_body,
        grid=(n // blk,),
        in_specs=[
            pl.BlockSpec((blk, D), lambda i: (i, 0)),
            pl.BlockSpec((D, D), lambda i: (0, 0)),
            pl.BlockSpec((D, D), lambda i: (0, 0)),
            pl.BlockSpec((D, D), lambda i: (0, 0)),
        ],
        out_specs=[
            pl.BlockSpec((blk, D), lambda i: (i, 0)),
            pl.BlockSpec((1, 1), lambda i: (0, 0), memory_space=pltpu.SMEM),
        ],
        out_shape=[
            jax.ShapeDtypeStruct((n, D), jnp.bfloat16),
            jax.ShapeDtypeStruct((1, 1), jnp.float32),
        ],
    )(x, mu, log_sigma, eps)


def _sc_propagate(support, src3, dst3, ew3, n_pad, n_chunks):
    rows_per_tile = n_pad // NS
    zr = 80
    mesh = plsc.VectorSubcoreMesh(core_axis_name="c", subcore_axis_name="s")

    @functools.partial(
        pl.kernel,
        out_type=jax.ShapeDtypeStruct((NC, n_pad, D), jnp.float32),
        mesh=mesh,
        scratch_types=[
            pltpu.VMEM((4, CHUNK), jnp.int32),
            pltpu.VMEM((4, CHUNK), jnp.int32),
            pltpu.VMEM((4, CHUNK), jnp.float32),
            pltpu.VMEM((2, CHUNK, 1, D // 2), jnp.int32),
            pltpu.VMEM((2, CHUNK, D), jnp.float32),
            pltpu.VMEM_SHARED((n_pad, D), jnp.float32),
            pltpu.SemaphoreType.DMA, pltpu.SemaphoreType.DMA,
            pltpu.SemaphoreType.DMA, pltpu.SemaphoreType.DMA,
            pltpu.SemaphoreType.DMA, pltpu.SemaphoreType.DMA,
            pltpu.SemaphoreType.DMA, pltpu.SemaphoreType.DMA,
            pltpu.SemaphoreType.DMA, pltpu.SemaphoreType.DMA,
        ],
    )
    def k(sup_hbm, src_hbm, dst_hbm, ew_hbm, out_hbm,
          src_r, dst_r, ew_r, rows_in, rows_out, acc_sh,
          gsem0, gsem1, ssem0, ssem1, rsem0, rsem1,
          lsem0, lsem1, esem0, esem1):
        c = lax.axis_index("c")
        s = lax.axis_index("s")
        wid = c * NS + s
        gsems = (gsem0, gsem1)
        ssems = (ssem0, ssem1)
        rsems = (rsem0, rsem1)
        lsems = (lsem0, lsem1)
        esems = (esem0, esem1)

        def zrow(i, _):
            for j in range(D // 16):
                rows_out[0, i, pl.ds(j * 16, 16)] = jnp.zeros((16,), jnp.float32)
            return 0
        lax.fori_loop(0, zr, zrow, 0)
        for t in range(rows_per_tile // zr):
            pltpu.sync_copy(rows_out.at[0, pl.ds(0, zr)],
                            acc_sh.at[pl.ds(s * rows_per_tile + t * zr, zr)])
        plsc.subcore_barrier()

        for slot in range(2):
            pltpu.sync_copy(src_hbm.at[wid, slot], src_r.at[slot])
            pltpu.async_copy(dst_hbm.at[wid, slot], dst_r.at[slot], lsems[slot])
            pltpu.async_copy(ew_hbm.at[wid, slot], ew_r.at[slot], esems[slot])
            pltpu.async_copy(sup_hbm.at[src_r.at[slot]],
                             rows_in.at[slot], gsems[slot])

        def pair_body(m, _):
            for slot in range(2):
                kk = 2 * m + slot
                r4 = lax.rem(kk, 4)
                r4n = lax.rem(kk + 2, 4)
                live = kk + 2 < n_chunks

                @pl.when(live)
                def _():
                    pltpu.async_copy(src_hbm.at[wid, kk + 2],
                                     src_r.at[r4n], rsems[slot])

                pltpu.make_async_copy(dst_hbm.at[wid, kk],
                                      dst_r.at[r4], lsems[slot]).wait()
                pltpu.make_async_copy(ew_hbm.at[wid, kk],
                                      ew_r.at[r4], esems[slot]).wait()
                pltpu.make_async_copy(sup_hbm.at[src_r.at[r4]],
                                      rows_in.at[slot], gsems[slot]).wait()

                @pl.when(m >= 1)
                def _():
                    pltpu.make_async_copy(rows_out.at[slot],
                                          acc_sh.at[dst_r.at[r4n]],
                                          ssems[slot]).wait()

                @pl.when(live)
                def _():
                    pltpu.async_copy(dst_hbm.at[wid, kk + 2],
                                     dst_r.at[r4n], lsems[slot])
                    pltpu.async_copy(ew_hbm.at[wid, kk + 2],
                                     ew_r.at[r4n], esems[slot])

                def scale(g, _):
                    base = g * 16
                    ewv = ew_r[r4, pl.ds(base, 16)]
                    himask = jnp.full((16,), -65536, jnp.int32)
                    sh16 = jnp.full((16,), 16, jnp.int32)
                    for t in range(16):
                        ewb = lax.broadcast(ewv[t], (16,))
                        for w in range(D // 32):
                            bits = rows_in[slot, base + t, 0, pl.ds(w * 16, 16)]
                            lo = lax.bitcast_convert_type(
                                lax.shift_left(bits, sh16), jnp.float32)
                            hi = lax.bitcast_convert_type(
                                lax.bitwise_and(bits, himask), jnp.float32)
                            rows_out[slot, base + t,
                                     pl.ds(w * 16, 16)] = lo * ewb
                            rows_out[slot, base + t,
                                     pl.ds(64 + w * 16, 16)] = hi * ewb
                    return 0
                lax.fori_loop(0, CHUNK // 16, scale, 0)

                pltpu.async_copy(rows_out.at[slot], acc_sh.at[dst_r.at[r4]],
                                 ssems[slot], add=True)

                @pl.when(live)
                def _():
                    pltpu.make_async_copy(src_hbm.at[wid, kk + 2],
                                          src_r.at[r4n], rsems[slot]).wait()
                    pltpu.async_copy(sup_hbm.at[src_r.at[r4n]],
                                     rows_in.at[slot], gsems[slot])
            return 0
        lax.fori_loop(0, n_chunks // 2, pair_body, 0)

        for slot in range(2):
            kk = n_chunks - 2 + slot
            pltpu.make_async_copy(rows_out.at[slot],
                                  acc_sh.at[dst_r.at[kk % 4]],
                                  ssems[slot]).wait()
        plsc.subcore_barrier()

        for t in range(rows_per_tile // zr):
            rsl = pl.ds(s * rows_per_tile + t * zr, zr)
            pltpu.sync_copy(acc_sh.at[rsl], rows_out.at[0, pl.ds(0, zr)])
            pltpu.sync_copy(rows_out.at[0, pl.ds(0, zr)], out_hbm.at[c].at[rsl])

    return k(support, src3, dst3, ew3)


def _tc_add_body(p_ref, out_ref):
    out_ref[...] = p_ref[0] + p_ref[1]


def _tc_add(partials):
    _, n, d = partials.shape
    blk = 2048
    return pl.pallas_call(
        _tc_add_body,
        grid=(n // blk,),
        in_specs=[pl.BlockSpec((NC, blk, d), lambda i: (0, i, 0))],
        out_specs=pl.BlockSpec((blk, d), lambda i: (i, 0)),
        out_shape=jax.ShapeDtypeStruct((n, d), jnp.float32),
    )(partials)


def kernel(x, edge_index, edge_weight, mu, log_sigma, eps):
    n_nodes = x.shape[0]

    perm = jnp.arange(D).reshape(2, D // 2).T.reshape(-1)
    support, kl = _tc_forward(x, mu[:, perm], log_sigma[:, perm], eps[:, perm])

    src = edge_index[0].astype(jnp.int32)
    dst = edge_index[1].astype(jnp.int32)
    ew = edge_weight.astype(jnp.float32)
    e = src.shape[0]
    n_chunks = -(-e // (NW * CHUNK))
    n_chunks += n_chunks % 2
    pad = NW * n_chunks * CHUNK - e
    src3 = jnp.pad(src, (0, pad)).reshape(NW, n_chunks, CHUNK)
    dst3 = jnp.pad(dst, (0, pad)).reshape(NW, n_chunks, CHUNK)
    ew3 = jnp.pad(ew, (0, pad)).reshape(NW, n_chunks, CHUNK)

    n_pad = NS * 640
    sup_i32 = jax.lax.bitcast_convert_type(
        support.reshape(n_nodes, D // 2, 2), jnp.int32).reshape(
            n_nodes, 1, D // 2)
    partials = _sc_propagate(sup_i32, src3, dst3, ew3, n_pad, n_chunks)
    out = _tc_add(partials)[:n_nodes]
    return out, kl[0, 0]


# speedup vs baseline: 1.0164x; 1.0164x over previous
import functools

import jax
import jax.numpy as jnp
from jax import lax
from jax.experimental import pallas as pl
from jax.experimental.pallas import tpu as pltpu
from jax.experimental.pallas import tpu_sc as plsc

D = 128
NC = 2
NS = 16
NW = NC * NS
CHUNK = 88


def _tc_forward_body(x_ref, mu_ref, ls_ref, eps_ref, sup_ref, kl_ref):
    ls = ls_ref[...]
    mu = mu_ref[...]
    sigma = jnp.exp(ls)
    w = mu + eps_ref[...] * sigma
    sup_ref[...] = jnp.dot(
        x_ref[...], w, preferred_element_type=jnp.float32
    ).astype(jnp.bfloat16)

    @pl.when(pl.program_id(0) == 0)
    def _():
        kl = 0.5 * (sigma * sigma + mu * mu - 2.0 * ls - 1.0)
        kl_ref[0, 0] = jnp.sum(kl)


def _tc_forward(x, mu, log_sigma, eps):
    n = x.shape[0]
    blk = 2000
    assert n % blk == 0
    return pl.pallas_call(
        _tc_forward_body,
        grid=(n // blk,),
        in_specs=[
            pl.BlockSpec((blk, D), lambda i: (i, 0)),
            pl.BlockSpec((D, D), lambda i: (0, 0)),
            pl.BlockSpec((D, D), lambda i: (0, 0)),
            pl.BlockSpec((D, D), lambda i: (0, 0)),
        ],
        out_specs=[
            pl.BlockSpec((blk, D), lambda i: (i, 0)),
            pl.BlockSpec((1, 1), lambda i: (0, 0), memory_space=pltpu.SMEM),
        ],
        out_shape=[
            jax.ShapeDtypeStruct((n, D), jnp.bfloat16),
            jax.ShapeDtypeStruct((1, 1), jnp.float32),
        ],
    )(x, mu, log_sigma, eps)


def _sc_propagate(support, src3, dst3, ew3, n_pad, n_chunks):
    rows_per_tile = n_pad // NS
    zr = 80
    mesh = plsc.VectorSubcoreMesh(core_axis_name="c", subcore_axis_name="s")

    @functools.partial(
        pl.kernel,
        out_type=jax.ShapeDtypeStruct((NC, n_pad, D), jnp.float32),
        mesh=mesh,
        scratch_types=[
            pltpu.VMEM((4, CHUNK), jnp.int32),
            pltpu.VMEM((4, CHUNK), jnp.int32),
            pltpu.VMEM((4, CHUNK), jnp.float32),
            pltpu.VMEM((2, CHUNK, 1, D // 2), jnp.int32),
            pltpu.VMEM((2, CHUNK, D), jnp.float32),
            pltpu.VMEM_SHARED((n_pad, D), jnp.float32),
            pltpu.SemaphoreType.DMA, pltpu.SemaphoreType.DMA,
            pltpu.SemaphoreType.DMA, pltpu.SemaphoreType.DMA,
            pltpu.SemaphoreType.DMA, pltpu.SemaphoreType.DMA,
            pltpu.SemaphoreType.DMA, pltpu.SemaphoreType.DMA,
            pltpu.SemaphoreType.DMA, pltpu.SemaphoreType.DMA,
        ],
    )
    def k(sup_hbm, src_hbm, dst_hbm, ew_hbm, out_hbm,
          src_r, dst_r, ew_r, rows_in, rows_out, acc_sh,
          gsem0, gsem1, ssem0, ssem1, rsem0, rsem1,
          lsem0, lsem1, esem0, esem1):
        c = lax.axis_index("c")
        s = lax.axis_index("s")
        wid = c * NS + s
        gsems = (gsem0, gsem1)
        ssems = (ssem0, ssem1)
        rsems = (rsem0, rsem1)
        lsems = (lsem0, lsem1)
        esems = (esem0, esem1)

        def zrow(i, _):
            for j in range(D // 16):
                rows_out[0, i, pl.ds(j * 16, 16)] = jnp.zeros((16,), jnp.float32)
            return 0
        lax.fori_loop(0, zr, zrow, 0)
        for t in range(rows_per_tile // zr):
            pltpu.sync_copy(rows_out.at[0, pl.ds(0, zr)],
                            acc_sh.at[pl.ds(s * rows_per_tile + t * zr, zr)])
        plsc.subcore_barrier()

        for slot in range(2):
            pltpu.sync_copy(src_hbm.at[wid, slot], src_r.at[slot])
            pltpu.async_copy(dst_hbm.at[wid, slot], dst_r.at[slot], lsems[slot])
            pltpu.async_copy(ew_hbm.at[wid, slot], ew_r.at[slot], esems[slot])
            pltpu.async_copy(sup_hbm.at[src_r.at[slot]],
                             rows_in.at[slot], gsems[slot])

        def pair_body(m, _):
            for slot in range(2):
                kk = 2 * m + slot
                r4 = lax.rem(kk, 4)
                r4n = lax.rem(kk + 2, 4)
                live = kk + 2 < n_chunks

                @pl.when(live)
                def _():
                    pltpu.async_copy(src_hbm.at[wid, kk + 2],
                                     src_r.at[r4n], rsems[slot])

                pltpu.make_async_copy(dst_hbm.at[wid, kk],
                                      dst_r.at[r4], lsems[slot]).wait()
                pltpu.make_async_copy(ew_hbm.at[wid, kk],
                                      ew_r.at[r4], esems[slot]).wait()
                pltpu.make_async_copy(sup_hbm.at[src_r.at[r4]],
                                      rows_in.at[slot], gsems[slot]).wait()

                @pl.when(m >= 1)
                def _():
                    pltpu.make_async_copy(rows_out.at[slot],
                                          acc_sh.at[dst_r.at[r4n]],
                                          ssems[slot]).wait()

                @pl.when(live)
                def _():
                    pltpu.async_copy(dst_hbm.at[wid, kk + 2],
                                     dst_r.at[r4n], lsems[slot])
                    pltpu.async_copy(ew_hbm.at[wid, kk + 2],
                                     ew_r.at[r4n], esems[slot])

                def scale(g, _):
                    base = g * 16
                    ewv = ew_r[r4, pl.ds(base, 16)]
                    himask = jnp.full((16,), -65536, jnp.int32)
                    sh16 = jnp.full((16,), 16, jnp.int32)
                    for t in range(16):
                        ewb = lax.broadcast(ewv[t], (16,))
                        for w in range(D // 32):
                            bits = rows_in[slot, base + t, 0, pl.ds(w * 16, 16)]
                            lo = lax.bitcast_convert_type(
                                lax.shift_left(bits, sh16), jnp.float32)
                            hi = lax.bitcast_convert_type(
                                lax.bitwise_and(bits, himask), jnp.float32)
                            rows_out[slot, base + t,
                                     pl.ds(w * 16, 16)] = lo * ewb
                            rows_out[slot, base + t,
                                     pl.ds(64 + w * 16, 16)] = hi * ewb
                    return 0
                lax.fori_loop(0, CHUNK // 16, scale, 0)

                pltpu.async_copy(rows_out.at[slot], acc_sh.at[dst_r.at[r4]],
                                 ssems[slot], add=True)

                @pl.when(live)
                def _():
                    pltpu.make_async_copy(src_hbm.at[wid, kk + 2],
                                          src_r.at[r4n], rsems[slot]).wait()
                    pltpu.async_copy(sup_hbm.at[src_r.at[r4n]],
                                     rows_in.at[slot], gsems[slot])
            return 0
        lax.fori_loop(0, n_chunks // 2, pair_body, 0)

        for slot in range(2):
            kk = n_chunks - 2 + slot
            pltpu.make_async_copy(rows_out.at[slot],
                                  acc_sh.at[dst_r.at[kk % 4]],
                                  ssems[slot]).wait()
        plsc.subcore_barrier()

        for t in range(rows_per_tile // zr):
            rsl = pl.ds(s * rows_per_tile + t * zr, zr)
            pltpu.sync_copy(acc_sh.at[rsl], out_hbm.at[c].at[rsl])

    return k(support, src3, dst3, ew3)


def _tc_add_body(p_ref, out_ref):
    out_ref[...] = p_ref[0] + p_ref[1]


def _tc_add(partials, n):
    d = partials.shape[-1]
    blk = 2000
    return pl.pallas_call(
        _tc_add_body,
        grid=(n // blk,),
        in_specs=[pl.BlockSpec((NC, blk, d), lambda i: (0, i, 0))],
        out_specs=pl.BlockSpec((blk, d), lambda i: (i, 0)),
        out_shape=jax.ShapeDtypeStruct((n, d), jnp.float32),
    )(partials)


def kernel(x, edge_index, edge_weight, mu, log_sigma, eps):
    n_nodes = x.shape[0]

    perm = jnp.arange(D).reshape(2, D // 2).T.reshape(-1)
    support, kl = _tc_forward(x, mu[:, perm], log_sigma[:, perm], eps[:, perm])

    src = edge_index[0].astype(jnp.int32)
    dst = edge_index[1].astype(jnp.int32)
    ew = edge_weight.astype(jnp.float32)
    e = src.shape[0]
    n_chunks = -(-e // (NW * CHUNK))
    n_chunks += n_chunks % 2
    pad = NW * n_chunks * CHUNK - e
    src3 = jnp.pad(src, (0, pad)).reshape(NW, n_chunks, CHUNK)
    dst3 = jnp.pad(dst, (0, pad)).reshape(NW, n_chunks, CHUNK)
    ew3 = jnp.pad(ew, (0, pad)).reshape(NW, n_chunks, CHUNK)

    n_pad = NS * 640
    sup_i32 = jax.lax.bitcast_convert_type(
        support.reshape(n_nodes, D // 2, 2), jnp.int32).reshape(
            n_nodes, 1, D // 2)
    partials = _sc_propagate(sup_i32, src3, dst3, ew3, n_pad, n_chunks)
    out = _tc_add(partials, n_nodes)
    return out, kl[0, 0]
